# baseline (speedup 1.0000x reference)
import jax
import jax.numpy as jnp
from jax.experimental import pallas as pl
from jax.experimental.pallas import tpu as pltpu

_STEADY = 2048
_RAMP = (256, 256, 512, 1024)
_NXBUF = 4
_NOBUF = 2


def _chunks(m):
    widths = list(_RAMP) + [_STEADY] * ((m - sum(_RAMP)) // _STEADY)
    assert sum(widths) == m
    offs, o = [], 0
    for w in widths:
        offs.append(o)
        o += w
    return list(zip(offs, widths))


def _body_factory(m, k, n):
    chunks = _chunks(m)
    nc = len(chunks)

    def body(xt_hbm, w_ref, o_hbm, xbuf, obuf, insems, outsems):
        def in_copy(ci):
            off, wd = chunks[ci]
            return pltpu.make_async_copy(
                xt_hbm.at[:, pl.ds(off, wd)],
                xbuf.at[ci % _NXBUF, :, pl.ds(0, wd)],
                insems.at[ci % _NXBUF],
            )

        def out_copy(ci):
            off, wd = chunks[ci]
            return pltpu.make_async_copy(
                obuf.at[ci % _NOBUF, pl.ds(0, wd), :],
                o_hbm.at[pl.ds(off, wd), :],
                outsems.at[ci % _NOBUF],
            )

        for ci in range(min(_NXBUF - 1, nc)):
            in_copy(ci).start()
        w = w_ref[...].astype(jnp.bfloat16)
        for ci, (off, wd) in enumerate(chunks):
            in_copy(ci).wait()
            x = xbuf[ci % _NXBUF, :, :wd].astype(jnp.bfloat16)
            acc = jax.lax.dot_general(
                x, w, (((0,), (0,)), ((), ())),
                preferred_element_type=jnp.float32)
            if ci >= _NOBUF:
                out_copy(ci - _NOBUF).wait()
            obuf[ci % _NOBUF, :wd, :] = acc
            out_copy(ci).start()
            nxt = ci + _NXBUF - 1
            if nxt < nc:
                in_copy(nxt).start()
        for ci in range(max(0, nc - _NOBUF), nc):
            out_copy(ci).wait()

    return body


def kernel(inputs, kernel):
    m, k = inputs.shape
    _, n = kernel.shape
    xt = inputs.T
    return pl.pallas_call(
        _body_factory(m, k, n),
        in_specs=[
            pl.BlockSpec(memory_space=pltpu.MemorySpace.HBM),
            pl.BlockSpec(memory_space=pltpu.MemorySpace.VMEM),
        ],
        out_specs=pl.BlockSpec(memory_space=pltpu.MemorySpace.HBM),
        out_shape=jax.ShapeDtypeStruct((m, n), jnp.float32),
        scratch_shapes=[
            pltpu.VMEM((_NXBUF, k, _STEADY), jnp.float32),
            pltpu.VMEM((_NOBUF, _STEADY, n), jnp.float32),
            pltpu.SemaphoreType.DMA((_NXBUF,)),
            pltpu.SemaphoreType.DMA((_NOBUF,)),
        ],
    )(xt, kernel)


# speedup vs baseline: 1.0836x; 1.0836x over previous
import jax
import jax.numpy as jnp
from jax.experimental import pallas as pl
from jax.experimental.pallas import tpu as pltpu

_BM = 2048


def _matmul_body(x1_ref, x2_ref, w_ref, o_ref):
    w = w_ref[...].astype(jnp.bfloat16)
    half = o_ref.shape[0] // 2
    x1 = x1_ref[...].astype(jnp.bfloat16)
    o_ref[:half, :] = jax.lax.dot_general(
        x1, w, (((0,), (0,)), ((), ())),
        preferred_element_type=jnp.float32)
    x2 = x2_ref[...].astype(jnp.bfloat16)
    o_ref[half:, :] = jax.lax.dot_general(
        x2, w, (((0,), (0,)), ((), ())),
        preferred_element_type=jnp.float32)


def kernel(inputs, kernel):
    m, k = inputs.shape
    _, n = kernel.shape
    bm = min(_BM, m)
    xt = inputs.T
    return pl.pallas_call(
        _matmul_body,
        grid=(m // bm,),
        in_specs=[
            pl.BlockSpec((k, bm // 2), lambda i: (0, 2 * i)),
            pl.BlockSpec((k, bm // 2), lambda i: (0, 2 * i + 1)),
            pl.BlockSpec((k, n), lambda i: (0, 0)),
        ],
        out_specs=pl.BlockSpec((bm, n), lambda i: (i, 0)),
        out_shape=jax.ShapeDtypeStruct((m, n), jnp.float32),
        compiler_params=pltpu.CompilerParams(
            dimension_semantics=("parallel",),
        ),
    )(xt, xt, kernel)
